# baseline (speedup 1.0000x reference)
import functools

import jax
import jax.numpy as jnp
from jax import lax
from jax.experimental import pallas as pl
from jax.experimental.pallas import tpu as pltpu
from jax.experimental.pallas import tpu_sc as plsc

BATCH = 16384
EMB = 64
H1 = 128
CHUNK = 128


def _gather3(uid2d, did2d, bid2d, user_table, device_table, brand_table):
    info = plsc.get_sparse_core_info()
    nc, ns = info.num_cores, info.num_subcores
    nw = nc * ns
    rows_per_w = BATCH // nw
    nchunk = rows_per_w // CHUNK

    mesh = plsc.VectorSubcoreMesh(core_axis_name="c", subcore_axis_name="s")

    @functools.partial(
        pl.kernel,
        mesh=mesh,
        compiler_params=pltpu.CompilerParams(use_tc_tiling_on_sc=False),
        out_type=(
            jax.ShapeDtypeStruct((BATCH, EMB), jnp.float32),
            jax.ShapeDtypeStruct((BATCH, EMB), jnp.float32),
            jax.ShapeDtypeStruct((BATCH, EMB), jnp.float32),
        ),
        scratch_types=[
            pltpu.VMEM((nchunk, CHUNK), jnp.int32),
            pltpu.VMEM((nchunk, CHUNK), jnp.int32),
            pltpu.VMEM((nchunk, CHUNK), jnp.int32),
            pltpu.VMEM((rows_per_w, EMB), jnp.float32),
            pltpu.VMEM((rows_per_w, EMB), jnp.float32),
            pltpu.VMEM((rows_per_w, EMB), jnp.float32),
            pltpu.SemaphoreType.DMA,
        ],
    )
    def gather_kernel(uid_hbm, did_hbm, bid_hbm, ut_hbm, dt_hbm, bt_hbm,
                      uo_hbm, do_hbm, bo_hbm,
                      uidx_v, didx_v, bidx_v, urows_v, drows_v, brows_v, sem):
        wid = lax.axis_index("s") * nc + lax.axis_index("c")
        rbase = wid * nchunk
        pltpu.sync_copy(uid_hbm.at[pl.ds(rbase, nchunk)], uidx_v)
        pltpu.sync_copy(did_hbm.at[pl.ds(rbase, nchunk)], didx_v)
        pltpu.sync_copy(bid_hbm.at[pl.ds(rbase, nchunk)], bidx_v)
        copies = []
        for j in range(nchunk):
            dst = pl.ds(j * CHUNK, CHUNK)
            copies.append(pltpu.async_copy(ut_hbm.at[uidx_v.at[j]], urows_v.at[dst], sem))
            copies.append(pltpu.async_copy(dt_hbm.at[didx_v.at[j]], drows_v.at[dst], sem))
            copies.append(pltpu.async_copy(bt_hbm.at[bidx_v.at[j]], brows_v.at[dst], sem))
        for c in copies:
            c.wait()
        base = wid * rows_per_w
        pltpu.sync_copy(urows_v, uo_hbm.at[pl.ds(base, rows_per_w)])
        pltpu.sync_copy(drows_v, do_hbm.at[pl.ds(base, rows_per_w)])
        pltpu.sync_copy(brows_v, bo_hbm.at[pl.ds(base, rows_per_w)])

    return gather_kernel(uid2d, did2d, bid2d, user_table, device_table, brand_table)


def _mlp_body(u_ref, d_ref, b_ref, w1_ref, b1_ref, w2_ref, b2_ref, w3_ref, b3_ref, o_ref):
    h = jnp.dot(u_ref[...], w1_ref[0:EMB, :], preferred_element_type=jnp.float32)
    h = h + jnp.dot(d_ref[...], w1_ref[EMB:2 * EMB, :], preferred_element_type=jnp.float32)
    h = h + jnp.dot(b_ref[...], w1_ref[2 * EMB:3 * EMB, :], preferred_element_type=jnp.float32)
    h = jnp.maximum(h + b1_ref[...], 0.0)
    h = jnp.maximum(jnp.dot(h, w2_ref[...], preferred_element_type=jnp.float32) + b2_ref[...], 0.0)
    o = jnp.dot(h, w3_ref[...], preferred_element_type=jnp.float32) + b3_ref[...]
    o_ref[...] = o


def _mlp(u, d, b, W1, b1, W2, b2, W3, b3):
    bb = 2048
    grid = (BATCH // bb,)
    return pl.pallas_call(
        _mlp_body,
        grid=grid,
        in_specs=[
            pl.BlockSpec((bb, EMB), lambda i: (i, 0)),
            pl.BlockSpec((bb, EMB), lambda i: (i, 0)),
            pl.BlockSpec((bb, EMB), lambda i: (i, 0)),
            pl.BlockSpec((3 * EMB, H1), lambda i: (0, 0)),
            pl.BlockSpec((1, H1), lambda i: (0, 0)),
            pl.BlockSpec((H1, EMB), lambda i: (0, 0)),
            pl.BlockSpec((1, EMB), lambda i: (0, 0)),
            pl.BlockSpec((EMB, 1), lambda i: (0, 0)),
            pl.BlockSpec((1, 1), lambda i: (0, 0)),
        ],
        out_specs=pl.BlockSpec((bb, 1), lambda i: (i, 0)),
        out_shape=jax.ShapeDtypeStruct((BATCH, 1), jnp.float32),
    )(u, d, b, W1, b1.reshape(1, H1), W2, b2.reshape(1, EMB), W3, b3.reshape(1, 1))


def kernel(user_ids, device_ids, brand_ids, user_table, device_table, brand_table,
           W1, b1, W2, b2, W3, b3):
    uid2d = user_ids.astype(jnp.int32).reshape(BATCH // CHUNK, CHUNK)
    did2d = device_ids.astype(jnp.int32).reshape(BATCH // CHUNK, CHUNK)
    bid2d = brand_ids.astype(jnp.int32).reshape(BATCH // CHUNK, CHUNK)
    u, d, b = _gather3(uid2d, did2d, bid2d, user_table, device_table, brand_table)
    out = _mlp(u, d, b, W1, b1, W2, b2, W3, b3)
    return out.reshape(BATCH)


# speedup vs baseline: 1.0256x; 1.0256x over previous
import functools

import jax
import jax.numpy as jnp
from jax import lax
from jax.experimental import pallas as pl
from jax.experimental.pallas import tpu as pltpu
from jax.experimental.pallas import tpu_sc as plsc

BATCH = 16384
EMB = 64
H1 = 128
CHUNK = 128


def _gather3(uid2d, did2d, bid2d, user_table, device_table, brand_table):
    info = plsc.get_sparse_core_info()
    nc, ns = info.num_cores, info.num_subcores
    nw = nc * ns
    rows_per_w = BATCH // nw
    nchunk = rows_per_w // CHUNK

    mesh = plsc.VectorSubcoreMesh(core_axis_name="c", subcore_axis_name="s")

    @functools.partial(
        pl.kernel,
        mesh=mesh,
        compiler_params=pltpu.CompilerParams(use_tc_tiling_on_sc=False),
        out_type=(
            jax.ShapeDtypeStruct((BATCH, 128), jnp.float32),
            jax.ShapeDtypeStruct((BATCH, 128), jnp.float32),
            jax.ShapeDtypeStruct((BATCH, 128), jnp.float32),
        ),
        scratch_types=[
            pltpu.VMEM((nchunk, CHUNK), jnp.int32),
            pltpu.VMEM((nchunk, CHUNK), jnp.int32),
            pltpu.VMEM((nchunk, CHUNK), jnp.int32),
            pltpu.VMEM((rows_per_w, EMB), jnp.float32),
            pltpu.VMEM((rows_per_w, EMB), jnp.float32),
            pltpu.VMEM((rows_per_w, EMB), jnp.float32),
            pltpu.SemaphoreType.DMA,
        ],
    )
    def gather_kernel(uid_hbm, did_hbm, bid_hbm, ut_hbm, dt_hbm, bt_hbm,
                      uo_hbm, do_hbm, bo_hbm,
                      uidx_v, didx_v, bidx_v, urows_v, drows_v, brows_v, sem):
        wid = lax.axis_index("s") * nc + lax.axis_index("c")
        rbase = wid * nchunk
        pltpu.sync_copy(uid_hbm.at[pl.ds(rbase, nchunk)], uidx_v)
        pltpu.sync_copy(did_hbm.at[pl.ds(rbase, nchunk)], didx_v)
        pltpu.sync_copy(bid_hbm.at[pl.ds(rbase, nchunk)], bidx_v)
        copies = []
        for j in range(nchunk):
            dst = pl.ds(j * CHUNK, CHUNK)
            copies.append(pltpu.async_copy(ut_hbm.at[uidx_v.at[j]], urows_v.at[dst], sem))
            copies.append(pltpu.async_copy(dt_hbm.at[didx_v.at[j]], drows_v.at[dst], sem))
            copies.append(pltpu.async_copy(bt_hbm.at[bidx_v.at[j]], brows_v.at[dst], sem))
        for c in copies:
            c.wait()
        base = wid * rows_per_w
        cols = pl.ds(0, EMB)
        pltpu.sync_copy(urows_v, uo_hbm.at[pl.ds(base, rows_per_w), cols])
        pltpu.sync_copy(drows_v, do_hbm.at[pl.ds(base, rows_per_w), cols])
        pltpu.sync_copy(brows_v, bo_hbm.at[pl.ds(base, rows_per_w), cols])

    return gather_kernel(uid2d, did2d, bid2d, user_table, device_table, brand_table)


def _mlp_body(u_ref, d_ref, b_ref, w1_ref, b1_ref, w2_ref, b2_ref, w3_ref, b3_ref, o_ref):
    h = jnp.dot(u_ref[:, 0:EMB], w1_ref[0:EMB, :], preferred_element_type=jnp.float32)
    h = h + jnp.dot(d_ref[:, 0:EMB], w1_ref[EMB:2 * EMB, :], preferred_element_type=jnp.float32)
    h = h + jnp.dot(b_ref[:, 0:EMB], w1_ref[2 * EMB:3 * EMB, :], preferred_element_type=jnp.float32)
    h = jnp.maximum(h + b1_ref[...], 0.0)
    h = jnp.maximum(jnp.dot(h, w2_ref[...], preferred_element_type=jnp.float32) + b2_ref[...], 0.0)
    o = jnp.dot(h, w3_ref[...], preferred_element_type=jnp.float32) + b3_ref[...]
    o_ref[...] = o


def _mlp(u, d, b, W1, b1, W2, b2, W3, b3):
    bb = 2048
    grid = (BATCH // bb,)
    return pl.pallas_call(
        _mlp_body,
        grid=grid,
        in_specs=[
            pl.BlockSpec((bb, 128), lambda i: (i, 0)),
            pl.BlockSpec((bb, 128), lambda i: (i, 0)),
            pl.BlockSpec((bb, 128), lambda i: (i, 0)),
            pl.BlockSpec((3 * EMB, H1), lambda i: (0, 0)),
            pl.BlockSpec((1, H1), lambda i: (0, 0)),
            pl.BlockSpec((H1, EMB), lambda i: (0, 0)),
            pl.BlockSpec((1, EMB), lambda i: (0, 0)),
            pl.BlockSpec((EMB, 1), lambda i: (0, 0)),
            pl.BlockSpec((1, 1), lambda i: (0, 0)),
        ],
        out_specs=pl.BlockSpec((bb, 1), lambda i: (i, 0)),
        out_shape=jax.ShapeDtypeStruct((BATCH, 1), jnp.float32),
    )(u, d, b, W1, b1.reshape(1, H1), W2, b2.reshape(1, EMB), W3, b3.reshape(1, 1))


def kernel(user_ids, device_ids, brand_ids, user_table, device_table, brand_table,
           W1, b1, W2, b2, W3, b3):
    uid2d = user_ids.astype(jnp.int32).reshape(BATCH // CHUNK, CHUNK)
    did2d = device_ids.astype(jnp.int32).reshape(BATCH // CHUNK, CHUNK)
    bid2d = brand_ids.astype(jnp.int32).reshape(BATCH // CHUNK, CHUNK)
    u, d, b = _gather3(uid2d, did2d, bid2d, user_table, device_table, brand_table)
    out = _mlp(u, d, b, W1, b1, W2, b2, W3, b3)
    return out.reshape(BATCH)
